# baseline (speedup 1.0000x reference)
import functools

import jax
import jax.numpy as jnp
from jax import lax
from jax.experimental import pallas as pl
from jax.experimental.pallas import tpu as pltpu
from jax.experimental.pallas import tpu_sc as plsc

NC = 2
NS = 16
N = 10000
E = 160000
C = 256
HALF = C // 2
NPAD = 10240
ZPT = NPAD // NS
CH = 128
EPS = 1e-5

_MESH = plsc.VectorSubcoreMesh(core_axis_name="c", subcore_axis_name="s",
                               num_cores=NC, num_subcores=NS)


DW = 128


def _deg_body(rowi, ones_h, zrows, out, acc, ones_v, ones_t, ridx_v, ridx_t):
    c = lax.axis_index("c")
    s = lax.axis_index("s")
    pltpu.sync_copy(zrows, acc.at[pl.ds(s * ZPT, ZPT)])
    pltpu.sync_copy(ones_h, ones_v)
    pltpu.sync_copy(ones_h.at[pl.ds(0, 8)], ones_t)
    plsc.subcore_barrier()

    ept = E // (NC * NS)
    full = ept // CH
    tail = ept - full * CH
    ebase = (c * NS + s) * ept

    def do(off, ridx, ones, sz):
        pltpu.sync_copy(rowi.at[pl.ds(off, sz)], ridx)
        pltpu.sync_copy(ones, acc.at[ridx], add=True)

    def body(i, carry):
        do(ebase + i * CH, ridx_v, ones_v, CH)
        return carry

    lax.fori_loop(0, full, body, 0)
    do(ebase + full * CH, ridx_t, ones_t, tail)
    plsc.subcore_barrier()
    pltpu.sync_copy(acc.at[pl.ds(s * ZPT, ZPT)], out.at[c, pl.ds(s * ZPT, ZPT)])


def _deg_call(row):
    ones_h = jnp.ones((CH, DW), jnp.float32)
    zrows = jnp.zeros((ZPT, DW), jnp.float32)
    fn = pl.kernel(
        _deg_body,
        out_type=jax.ShapeDtypeStruct((NC, NPAD, DW), jnp.float32),
        mesh=_MESH,
        scratch_types=[
            pltpu.VMEM_SHARED((NPAD, DW), jnp.float32),
            pltpu.VMEM((CH, DW), jnp.float32),
            pltpu.VMEM((8, DW), jnp.float32),
            pltpu.VMEM((CH,), jnp.int32),
            pltpu.VMEM((8,), jnp.int32),
        ],
    )
    return fn(row, ones_h, zrows)


_GRP = 8
_NCHUNK = 1280
EPAD = _NCHUNK * CH
_CPT = _NCHUNK // NS
_NG = _CPT // _GRP


def _prop_body(row2, col2, hp, zrows, out, acc,
               rows2, ridx_g, cidx_g, gsem2, ssem2):
    c = lax.axis_index("c")
    s = lax.axis_index("s")
    coff = c * NPAD
    pltpu.sync_copy(zrows, acc.at[pl.ds(s * ZPT, ZPT)])
    plsc.subcore_barrier()

    cbase = s * _CPT

    def wait_g(b):
        pltpu.make_async_copy(hp.at[ridx_g.at[0]], rows2[b], gsem2[b]).wait()

    def wait_s(b):
        pltpu.make_async_copy(rows2[b], acc.at[cidx_g.at[0]], ssem2[b]).wait()

    def group(g, carry):
        goff = cbase + g * _GRP
        pltpu.sync_copy(row2.at[pl.ds(goff, _GRP)], ridx_g)
        pltpu.sync_copy(col2.at[pl.ds(goff, _GRP)], cidx_g)
        for r in range(_GRP):
            for k in range(CH // 16):
                sl = pl.ds(k * 16, 16)
                ridx_g[r, sl] = ridx_g[r, sl] + coff
        @pl.when(g > 0)
        def _():
            wait_s(0)
        pltpu.async_copy(hp.at[ridx_g.at[0]], rows2[0], gsem2[0])
        for k in range(_GRP):
            b = k % 2
            if k + 1 < _GRP:
                nb = (k + 1) % 2
                if k + 1 >= 2:
                    wait_s(nb)
                else:
                    @pl.when(g > 0)
                    def _():
                        wait_s(nb)
                pltpu.async_copy(hp.at[ridx_g.at[k + 1]], rows2[nb], gsem2[nb])
            wait_g(b)
            pltpu.async_copy(rows2[b], acc.at[cidx_g.at[k]], ssem2[b], add=True)
        return carry

    lax.fori_loop(0, _NG, group, 0)
    wait_s(0)
    wait_s(1)
    plsc.subcore_barrier()
    pltpu.sync_copy(acc.at[pl.ds(s * ZPT, ZPT)], out.at[c, pl.ds(s * ZPT, ZPT)])


def _prop_call(hp_flat, row2, col2):
    zrows = jnp.zeros((ZPT, HALF), jnp.float32)
    fn = pl.kernel(
        _prop_body,
        out_type=jax.ShapeDtypeStruct((NC, NPAD, HALF), jnp.float32),
        mesh=_MESH,
        scratch_types=[
            pltpu.VMEM_SHARED((NPAD, HALF), jnp.float32),
            [pltpu.VMEM((CH, HALF), jnp.float32)] * 2,
            pltpu.VMEM((_GRP, CH), jnp.int32),
            pltpu.VMEM((_GRP, CH), jnp.int32),
            [pltpu.SemaphoreType.DMA] * 2,
            [pltpu.SemaphoreType.DMA] * 2,
        ],
    )
    return fn(row2, col2, hp_flat, zrows)


_BN = 400
_NB = N // _BN


def _diw_block(deg_ref):
    deg = jnp.sum(deg_ref[...], axis=1, keepdims=True)
    return jnp.where(deg > 0, lax.rsqrt(jnp.maximum(deg, 1.0)), 0.0)


def _e1_body(deg_ref, x_ref, out_ref):
    d2 = _diw_block(deg_ref)
    out_ref[0] = d2 * x_ref[:, :HALF]
    out_ref[1] = d2 * x_ref[:, HALF:]


def _e1_call(deg2, x):
    return pl.pallas_call(
        _e1_body,
        grid=(_NB,),
        in_specs=[pl.BlockSpec((_BN, 2), lambda i: (i, 0)),
                  pl.BlockSpec((_BN, C), lambda i: (i, 0))],
        out_specs=pl.BlockSpec((NC, _BN, HALF), lambda i: (0, i, 0)),
        out_shape=jax.ShapeDtypeStruct((NC, NPAD, HALF), jnp.float32),
    )(deg2, x)


def _e2_body(deg_ref, a_ref, out_ref):
    deg = jnp.sum(deg_ref[...], axis=1, keepdims=True)
    w = jnp.where(deg > 0, -1.0 / jnp.maximum(deg, 1.0), 0.0)
    out_ref[0] = w * a_ref[0]
    out_ref[1] = w * a_ref[1]


def _e2_call(deg2, acc1):
    return pl.pallas_call(
        _e2_body,
        grid=(_NB,),
        in_specs=[pl.BlockSpec((_BN, 2), lambda i: (i, 0)),
                  pl.BlockSpec((NC, _BN, HALF), lambda i: (0, i, 0))],
        out_specs=pl.BlockSpec((NC, _BN, HALF), lambda i: (0, i, 0)),
        out_shape=jax.ShapeDtypeStruct((NC, NPAD, HALF), jnp.float32),
    )(deg2, acc1)


def _d1_body(deg_ref, x_ref, a1_ref, a2_ref, w_ref, b_ref,
             y_ref, st_ref, s_acc, q_acc):
    i = pl.program_id(0)
    d2 = _diw_block(deg_ref)
    x = x_ref[...]
    t1 = -d2 * jnp.concatenate([a1_ref[0], a1_ref[1]], axis=1)
    t2 = -2.0 * d2 * jnp.concatenate([a2_ref[0], a2_ref[1]], axis=1) - x
    y = jnp.dot(x, w_ref[0], preferred_element_type=jnp.float32)
    y = y + jnp.dot(t1, w_ref[1], preferred_element_type=jnp.float32)
    y = y + jnp.dot(t2, w_ref[2], preferred_element_type=jnp.float32)
    y = y + b_ref[...][None, :]
    y_ref[...] = y

    @pl.when(i == 0)
    def _():
        s_acc[...] = jnp.zeros_like(s_acc)
        q_acc[...] = jnp.zeros_like(q_acc)

    s_acc[...] += jnp.sum(y, axis=0, keepdims=True)
    q_acc[...] += jnp.sum(y * y, axis=0, keepdims=True)

    @pl.when(i == _NB - 1)
    def _():
        st_ref[0] = s_acc[0]
        st_ref[1] = q_acc[0]


def _d1_call(deg2, x, acc1, acc2, W, bias):
    return pl.pallas_call(
        _d1_body,
        grid=(_NB,),
        in_specs=[pl.BlockSpec((_BN, 2), lambda i: (i, 0)),
                  pl.BlockSpec((_BN, C), lambda i: (i, 0)),
                  pl.BlockSpec((NC, _BN, HALF), lambda i: (0, i, 0)),
                  pl.BlockSpec((NC, _BN, HALF), lambda i: (0, i, 0)),
                  pl.BlockSpec((3, C, C), lambda i: (0, 0, 0)),
                  pl.BlockSpec((C,), lambda i: (0,))],
        out_specs=[pl.BlockSpec((_BN, C), lambda i: (i, 0)),
                   pl.BlockSpec((2, C), lambda i: (0, 0))],
        out_shape=[jax.ShapeDtypeStruct((N, C), jnp.float32),
                   jax.ShapeDtypeStruct((2, C), jnp.float32)],
        scratch_shapes=[pltpu.VMEM((1, C), jnp.float32),
                        pltpu.VMEM((1, C), jnp.float32)],
    )(deg2, x, acc1, acc2, W, bias)


def _d2_body(st_ref, x_ref, y_ref, g_ref, bt_ref, o_ref):
    mean = st_ref[0] / N
    var = st_ref[1] / N - mean * mean
    scale = (lax.rsqrt(var + EPS) * g_ref[...])[None, :]
    o_ref[...] = jnp.maximum(
        (y_ref[...] - mean[None, :]) * scale + bt_ref[...][None, :] + x_ref[...],
        0.0)


def _d2_call(st, x, y, gamma, beta):
    return pl.pallas_call(
        _d2_body,
        grid=(_NB,),
        in_specs=[pl.BlockSpec((2, C), lambda i: (0, 0)),
                  pl.BlockSpec((_BN, C), lambda i: (i, 0)),
                  pl.BlockSpec((_BN, C), lambda i: (i, 0)),
                  pl.BlockSpec((C,), lambda i: (0,)),
                  pl.BlockSpec((C,), lambda i: (0,))],
        out_specs=pl.BlockSpec((_BN, C), lambda i: (i, 0)),
        out_shape=jax.ShapeDtypeStruct((N, C), jnp.float32),
    )(st, x, y, gamma, beta)


def kernel(x, edge_index, W, bias, gamma, beta):
    row = edge_index[0]
    col = edge_index[1]
    npd = EPAD - E
    row2 = jnp.concatenate([row, jnp.zeros((npd,), jnp.int32)]).reshape(_NCHUNK, CH)
    cpad = N + jnp.arange(npd, dtype=jnp.int32) % (NPAD - N)
    col2 = jnp.concatenate([col, cpad]).reshape(_NCHUNK, CH)
    degs = _deg_call(row)
    deg2 = degs[:, :N, 0].T
    hp0 = _e1_call(deg2, x).reshape(NC * NPAD, HALF)
    acc1 = _prop_call(hp0, row2, col2)
    hp1 = _e2_call(deg2, acc1).reshape(NC * NPAD, HALF)
    acc2 = _prop_call(hp1, row2, col2)
    y, st = _d1_call(deg2, x, acc1, acc2, W, bias)
    return _d2_call(st, x, y, gamma, beta)


# speedup vs baseline: 1.0491x; 1.0491x over previous
import functools

import jax
import jax.numpy as jnp
from jax import lax
from jax.experimental import pallas as pl
from jax.experimental.pallas import tpu as pltpu
from jax.experimental.pallas import tpu_sc as plsc

NC = 2
NS = 16
N = 10000
E = 160000
C = 256
HALF = C // 2
NPAD = 10240
ZPT = NPAD // NS
CH = 128
EPS = 1e-5

_MESH = plsc.VectorSubcoreMesh(core_axis_name="c", subcore_axis_name="s",
                               num_cores=NC, num_subcores=NS)


DW = 128


def _deg_body(rowi, ones_h, zrows, out, acc, ones_v, ones_t, ridx_v, ridx_t):
    c = lax.axis_index("c")
    s = lax.axis_index("s")
    pltpu.sync_copy(zrows, acc.at[pl.ds(s * ZPT, ZPT)])
    pltpu.sync_copy(ones_h, ones_v)
    pltpu.sync_copy(ones_h.at[pl.ds(0, 8)], ones_t)
    plsc.subcore_barrier()

    ept = E // (NC * NS)
    full = ept // CH
    tail = ept - full * CH
    ebase = (c * NS + s) * ept

    def do(off, ridx, ones, sz):
        pltpu.sync_copy(rowi.at[pl.ds(off, sz)], ridx)
        pltpu.sync_copy(ones, acc.at[ridx], add=True)

    def body(i, carry):
        do(ebase + i * CH, ridx_v, ones_v, CH)
        return carry

    lax.fori_loop(0, full, body, 0)
    do(ebase + full * CH, ridx_t, ones_t, tail)
    plsc.subcore_barrier()
    pltpu.sync_copy(acc.at[pl.ds(s * ZPT, ZPT)], out.at[c, pl.ds(s * ZPT, ZPT)])


def _deg_call(row):
    ones_h = jnp.ones((CH, DW), jnp.float32)
    zrows = jnp.zeros((ZPT, DW), jnp.float32)
    fn = pl.kernel(
        _deg_body,
        out_type=jax.ShapeDtypeStruct((NC, NPAD, DW), jnp.float32),
        mesh=_MESH,
        scratch_types=[
            pltpu.VMEM_SHARED((NPAD, DW), jnp.float32),
            pltpu.VMEM((CH, DW), jnp.float32),
            pltpu.VMEM((8, DW), jnp.float32),
            pltpu.VMEM((CH,), jnp.int32),
            pltpu.VMEM((8,), jnp.int32),
        ],
    )
    return fn(row, ones_h, zrows)


_GRP = 8
_NCHUNK = 1280
EPAD = _NCHUNK * CH
_CPT = _NCHUNK // NS
_NG = _CPT // _GRP


def _prop_body(rowi, coli, hp, zrows, out, acc,
               rows2, ridx2, cidx2, gsem2, ssem2):
    c = lax.axis_index("c")
    s = lax.axis_index("s")
    coff = c * NPAD
    pltpu.sync_copy(zrows, acc.at[pl.ds(s * ZPT, ZPT)])
    plsc.subcore_barrier()

    ebase = s * (_CPT * CH)
    npair = _CPT // 2

    def loadidx(off, d):
        pltpu.sync_copy(rowi.at[pl.ds(off, CH)], ridx2[d])
        pltpu.sync_copy(coli.at[pl.ds(off, CH)], cidx2[d])
        for k in range(CH // 16):
            sl = pl.ds(k * 16, 16)
            ridx2[d][sl] = ridx2[d][sl] + coff

    for d in range(2):
        loadidx(ebase + d * CH, d)
        pltpu.async_copy(hp.at[ridx2[d]], rows2[d], gsem2[d])

    def body(j, carry):
        base_next = ebase + (j + 1) * 2 * CH
        for d in range(2):
            pltpu.make_async_copy(hp.at[ridx2[d]], rows2[d], gsem2[d]).wait()
            pltpu.async_copy(rows2[d], acc.at[cidx2[d]], ssem2[d], add=True)

        @pl.when(j < npair - 1)
        def _():
            for d in range(2):
                pltpu.make_async_copy(rows2[d], acc.at[cidx2[d]], ssem2[d]).wait()
                loadidx(base_next + d * CH, d)
                pltpu.async_copy(hp.at[ridx2[d]], rows2[d], gsem2[d])

        return carry

    lax.fori_loop(0, npair, body, 0)
    for d in range(2):
        pltpu.make_async_copy(rows2[d], acc.at[cidx2[d]], ssem2[d]).wait()

    plsc.subcore_barrier()
    pltpu.sync_copy(acc.at[pl.ds(s * ZPT, ZPT)], out.at[c, pl.ds(s * ZPT, ZPT)])


def _prop_call(hp_flat, rowp, colp):
    zrows = jnp.zeros((ZPT, HALF), jnp.float32)
    fn = pl.kernel(
        _prop_body,
        out_type=jax.ShapeDtypeStruct((NC, NPAD, HALF), jnp.float32),
        mesh=_MESH,
        scratch_types=[
            pltpu.VMEM_SHARED((NPAD, HALF), jnp.float32),
            [pltpu.VMEM((CH, HALF), jnp.float32)] * 2,
            [pltpu.VMEM((CH,), jnp.int32)] * 2,
            [pltpu.VMEM((CH,), jnp.int32)] * 2,
            [pltpu.SemaphoreType.DMA] * 2,
            [pltpu.SemaphoreType.DMA] * 2,
        ],
    )
    return fn(rowp, colp, hp_flat, zrows)


_BN = 400
_NB = N // _BN


def _diw_block(deg_ref):
    deg = jnp.sum(deg_ref[...], axis=1, keepdims=True)
    return jnp.where(deg > 0, lax.rsqrt(jnp.maximum(deg, 1.0)), 0.0)


def _e1_body(deg_ref, x_ref, out_ref):
    d2 = _diw_block(deg_ref)
    out_ref[0] = d2 * x_ref[:, :HALF]
    out_ref[1] = d2 * x_ref[:, HALF:]


def _e1_call(deg2, x):
    return pl.pallas_call(
        _e1_body,
        grid=(_NB,),
        in_specs=[pl.BlockSpec((_BN, 2), lambda i: (i, 0)),
                  pl.BlockSpec((_BN, C), lambda i: (i, 0))],
        out_specs=pl.BlockSpec((NC, _BN, HALF), lambda i: (0, i, 0)),
        out_shape=jax.ShapeDtypeStruct((NC, NPAD, HALF), jnp.float32),
    )(deg2, x)


def _e2_body(deg_ref, a_ref, out_ref):
    deg = jnp.sum(deg_ref[...], axis=1, keepdims=True)
    w = jnp.where(deg > 0, -1.0 / jnp.maximum(deg, 1.0), 0.0)
    out_ref[0] = w * a_ref[0]
    out_ref[1] = w * a_ref[1]


def _e2_call(deg2, acc1):
    return pl.pallas_call(
        _e2_body,
        grid=(_NB,),
        in_specs=[pl.BlockSpec((_BN, 2), lambda i: (i, 0)),
                  pl.BlockSpec((NC, _BN, HALF), lambda i: (0, i, 0))],
        out_specs=pl.BlockSpec((NC, _BN, HALF), lambda i: (0, i, 0)),
        out_shape=jax.ShapeDtypeStruct((NC, NPAD, HALF), jnp.float32),
    )(deg2, acc1)


def _d1_body(deg_ref, x_ref, a1_ref, a2_ref, w_ref, b_ref,
             y_ref, st_ref, s_acc, q_acc):
    i = pl.program_id(0)
    d2 = _diw_block(deg_ref)
    x = x_ref[...]
    t1 = -d2 * jnp.concatenate([a1_ref[0], a1_ref[1]], axis=1)
    t2 = -2.0 * d2 * jnp.concatenate([a2_ref[0], a2_ref[1]], axis=1) - x
    y = jnp.dot(x, w_ref[0], preferred_element_type=jnp.float32)
    y = y + jnp.dot(t1, w_ref[1], preferred_element_type=jnp.float32)
    y = y + jnp.dot(t2, w_ref[2], preferred_element_type=jnp.float32)
    y = y + b_ref[...][None, :]
    y_ref[...] = y

    @pl.when(i == 0)
    def _():
        s_acc[...] = jnp.zeros_like(s_acc)
        q_acc[...] = jnp.zeros_like(q_acc)

    s_acc[...] += jnp.sum(y, axis=0, keepdims=True)
    q_acc[...] += jnp.sum(y * y, axis=0, keepdims=True)

    @pl.when(i == _NB - 1)
    def _():
        st_ref[0] = s_acc[0]
        st_ref[1] = q_acc[0]


def _d1_call(deg2, x, acc1, acc2, W, bias):
    return pl.pallas_call(
        _d1_body,
        grid=(_NB,),
        in_specs=[pl.BlockSpec((_BN, 2), lambda i: (i, 0)),
                  pl.BlockSpec((_BN, C), lambda i: (i, 0)),
                  pl.BlockSpec((NC, _BN, HALF), lambda i: (0, i, 0)),
                  pl.BlockSpec((NC, _BN, HALF), lambda i: (0, i, 0)),
                  pl.BlockSpec((3, C, C), lambda i: (0, 0, 0)),
                  pl.BlockSpec((C,), lambda i: (0,))],
        out_specs=[pl.BlockSpec((_BN, C), lambda i: (i, 0)),
                   pl.BlockSpec((2, C), lambda i: (0, 0))],
        out_shape=[jax.ShapeDtypeStruct((N, C), jnp.float32),
                   jax.ShapeDtypeStruct((2, C), jnp.float32)],
        scratch_shapes=[pltpu.VMEM((1, C), jnp.float32),
                        pltpu.VMEM((1, C), jnp.float32)],
    )(deg2, x, acc1, acc2, W, bias)


def _d2_body(st_ref, x_ref, y_ref, g_ref, bt_ref, o_ref):
    mean = st_ref[0] / N
    var = st_ref[1] / N - mean * mean
    scale = (lax.rsqrt(var + EPS) * g_ref[...])[None, :]
    o_ref[...] = jnp.maximum(
        (y_ref[...] - mean[None, :]) * scale + bt_ref[...][None, :] + x_ref[...],
        0.0)


def _d2_call(st, x, y, gamma, beta):
    return pl.pallas_call(
        _d2_body,
        grid=(_NB,),
        in_specs=[pl.BlockSpec((2, C), lambda i: (0, 0)),
                  pl.BlockSpec((_BN, C), lambda i: (i, 0)),
                  pl.BlockSpec((_BN, C), lambda i: (i, 0)),
                  pl.BlockSpec((C,), lambda i: (0,)),
                  pl.BlockSpec((C,), lambda i: (0,))],
        out_specs=pl.BlockSpec((_BN, C), lambda i: (i, 0)),
        out_shape=jax.ShapeDtypeStruct((N, C), jnp.float32),
    )(st, x, y, gamma, beta)


def kernel(x, edge_index, W, bias, gamma, beta):
    row = edge_index[0]
    col = edge_index[1]
    npd = EPAD - E
    rowp = jnp.concatenate([row, jnp.zeros((npd,), jnp.int32)])
    colp = jnp.concatenate([col, N + jnp.arange(npd, dtype=jnp.int32) % (NPAD - N)])
    degs = _deg_call(row)
    deg2 = degs[:, :N, 0].T
    hp0 = _e1_call(deg2, x).reshape(NC * NPAD, HALF)
    acc1 = _prop_call(hp0, rowp, colp)
    hp1 = _e2_call(deg2, acc1).reshape(NC * NPAD, HALF)
    acc2 = _prop_call(hp1, rowp, colp)
    y, st = _d1_call(deg2, x, acc1, acc2, W, bias)
    return _d2_call(st, x, y, gamma, beta)


# speedup vs baseline: 1.1124x; 1.0603x over previous
import functools

import jax
import jax.numpy as jnp
from jax import lax
from jax.experimental import pallas as pl
from jax.experimental.pallas import tpu as pltpu
from jax.experimental.pallas import tpu_sc as plsc

NC = 2
NS = 16
N = 10000
E = 160000
C = 256
HALF = C // 2
NPAD = 10240
ZPT = NPAD // NS
CH = 128
EPS = 1e-5

_MESH = plsc.VectorSubcoreMesh(core_axis_name="c", subcore_axis_name="s",
                               num_cores=NC, num_subcores=NS)


DW = 128


def _deg_body(rowi, ones_h, zrows, out, acc, ones_v, ones_t, ridx_v, ridx_t):
    c = lax.axis_index("c")
    s = lax.axis_index("s")
    pltpu.sync_copy(zrows, acc.at[pl.ds(s * ZPT, ZPT)])
    pltpu.sync_copy(ones_h, ones_v)
    pltpu.sync_copy(ones_h.at[pl.ds(0, 8)], ones_t)
    plsc.subcore_barrier()

    ept = E // (NC * NS)
    full = ept // CH
    tail = ept - full * CH
    ebase = (c * NS + s) * ept

    def do(off, ridx, ones, sz):
        pltpu.sync_copy(rowi.at[pl.ds(off, sz)], ridx)
        pltpu.sync_copy(ones, acc.at[ridx], add=True)

    def body(i, carry):
        do(ebase + i * CH, ridx_v, ones_v, CH)
        return carry

    lax.fori_loop(0, full, body, 0)
    do(ebase + full * CH, ridx_t, ones_t, tail)
    plsc.subcore_barrier()
    pltpu.sync_copy(acc.at[pl.ds(s * ZPT, ZPT)], out.at[c, pl.ds(s * ZPT, ZPT)])


def _deg_call(row):
    ones_h = jnp.ones((CH, DW), jnp.float32)
    zrows = jnp.zeros((ZPT, DW), jnp.float32)
    fn = pl.kernel(
        _deg_body,
        out_type=jax.ShapeDtypeStruct((NC, NPAD, DW), jnp.float32),
        mesh=_MESH,
        scratch_types=[
            pltpu.VMEM_SHARED((NPAD, DW), jnp.float32),
            pltpu.VMEM((CH, DW), jnp.float32),
            pltpu.VMEM((8, DW), jnp.float32),
            pltpu.VMEM((CH,), jnp.int32),
            pltpu.VMEM((8,), jnp.int32),
        ],
    )
    return fn(row, ones_h, zrows)


_GRP = 8
_NCHUNK = 1280
EPAD = _NCHUNK * CH
_CPT = _NCHUNK // NS
_NG = _CPT // _GRP


def _prop_body(rowi, coli, hp, zrows, out, acc,
               rows2, ridx2, cidx2, gsem2, ssem2):
    c = lax.axis_index("c")
    s = lax.axis_index("s")
    coff = c * NPAD
    pltpu.sync_copy(zrows, acc.at[pl.ds(s * ZPT, ZPT)])
    plsc.subcore_barrier()

    ebase = s * (_CPT * CH)
    npair = _CPT // 2

    def loadidx(off, d):
        pltpu.sync_copy(rowi.at[pl.ds(off, CH)], ridx2[d])
        pltpu.sync_copy(coli.at[pl.ds(off, CH)], cidx2[d])
        for k in range(CH // 16):
            sl = pl.ds(k * 16, 16)
            ridx2[d][sl] = ridx2[d][sl] + coff

    for d in range(2):
        loadidx(ebase + d * CH, d)
        pltpu.async_copy(hp.at[ridx2[d]], rows2[d], gsem2[d])

    def body(j, carry):
        base_next = ebase + (j + 1) * 2 * CH
        for d in range(2):
            pltpu.make_async_copy(hp.at[ridx2[d]], rows2[d], gsem2[d]).wait()
            pltpu.async_copy(rows2[d], acc.at[cidx2[d]], ssem2[d], add=True)

        @pl.when(j < npair - 1)
        def _():
            for d in range(2):
                pltpu.make_async_copy(rows2[d], acc.at[cidx2[d]], ssem2[d]).wait()
                loadidx(base_next + d * CH, d)
                pltpu.async_copy(hp.at[ridx2[d]], rows2[d], gsem2[d])

        return carry

    lax.fori_loop(0, npair, body, 0)
    for d in range(2):
        pltpu.make_async_copy(rows2[d], acc.at[cidx2[d]], ssem2[d]).wait()

    plsc.subcore_barrier()
    pltpu.sync_copy(acc.at[pl.ds(s * ZPT, ZPT)], out.at[c, pl.ds(s * ZPT, ZPT)])


def _prop_call(hp_flat, rowp, colp):
    zrows = jnp.zeros((ZPT, HALF), jnp.float32)
    fn = pl.kernel(
        _prop_body,
        out_type=jax.ShapeDtypeStruct((NC, NPAD, HALF), jnp.float32),
        mesh=_MESH,
        scratch_types=[
            pltpu.VMEM_SHARED((NPAD, HALF), jnp.float32),
            [pltpu.VMEM((CH, HALF), jnp.float32)] * 2,
            [pltpu.VMEM((CH,), jnp.int32)] * 2,
            [pltpu.VMEM((CH,), jnp.int32)] * 2,
            [pltpu.SemaphoreType.DMA] * 2,
            [pltpu.SemaphoreType.DMA] * 2,
        ],
    )
    return fn(rowp, colp, hp_flat, zrows)


_BN = 400
_NB = N // _BN


def _diw_block(deg_ref):
    deg = jnp.sum(deg_ref[...], axis=1, keepdims=True)
    return jnp.where(deg > 0, lax.rsqrt(jnp.maximum(deg, 1.0)), 0.0)


def _e1_body(deg_ref, x_ref, out_ref):
    d2 = _diw_block(deg_ref)
    out_ref[0] = d2 * x_ref[:, :HALF]
    out_ref[1] = d2 * x_ref[:, HALF:]


def _e1_call(deg2, x):
    return pl.pallas_call(
        _e1_body,
        grid=(_NB,),
        in_specs=[pl.BlockSpec((_BN, 2), lambda i: (i, 0)),
                  pl.BlockSpec((_BN, C), lambda i: (i, 0))],
        out_specs=pl.BlockSpec((NC, _BN, HALF), lambda i: (0, i, 0)),
        out_shape=jax.ShapeDtypeStruct((NC, NPAD, HALF), jnp.float32),
    )(deg2, x)


def _e2_body(deg_ref, a_ref, out_ref):
    deg = jnp.sum(deg_ref[...], axis=1, keepdims=True)
    w = jnp.where(deg > 0, -1.0 / jnp.maximum(deg, 1.0), 0.0)
    out_ref[0] = w * a_ref[0]
    out_ref[1] = w * a_ref[1]


def _e2_call(deg2, acc1):
    return pl.pallas_call(
        _e2_body,
        grid=(_NB,),
        in_specs=[pl.BlockSpec((_BN, 2), lambda i: (i, 0)),
                  pl.BlockSpec((NC, _BN, HALF), lambda i: (0, i, 0))],
        out_specs=pl.BlockSpec((NC, _BN, HALF), lambda i: (0, i, 0)),
        out_shape=jax.ShapeDtypeStruct((NC, NPAD, HALF), jnp.float32),
    )(deg2, acc1)


def _d1_body(deg_ref, x_ref, a1_ref, a2_ref, w_ref, b_ref,
             y_ref, st_ref, s_acc, q_acc):
    i = pl.program_id(0)
    d2 = _diw_block(deg_ref)
    x = x_ref[...]
    t1 = -d2 * jnp.concatenate([a1_ref[0], a1_ref[1]], axis=1)
    t2 = -2.0 * d2 * jnp.concatenate([a2_ref[0], a2_ref[1]], axis=1) - x
    y = jnp.dot(x, w_ref[0], preferred_element_type=jnp.float32)
    y = y + jnp.dot(t1, w_ref[1], preferred_element_type=jnp.float32)
    y = y + jnp.dot(t2, w_ref[2], preferred_element_type=jnp.float32)
    y = y + b_ref[...][None, :]
    y_ref[...] = y

    @pl.when(i == 0)
    def _():
        s_acc[...] = jnp.zeros_like(s_acc)
        q_acc[...] = jnp.zeros_like(q_acc)

    s_acc[...] += jnp.sum(y, axis=0, keepdims=True)
    q_acc[...] += jnp.sum(y * y, axis=0, keepdims=True)

    @pl.when(i == _NB - 1)
    def _():
        st_ref[0] = s_acc[0]
        st_ref[1] = q_acc[0]


def _d1_call(deg2, x, acc1, acc2, W, bias):
    return pl.pallas_call(
        _d1_body,
        grid=(_NB,),
        in_specs=[pl.BlockSpec((_BN, 2), lambda i: (i, 0)),
                  pl.BlockSpec((_BN, C), lambda i: (i, 0)),
                  pl.BlockSpec((NC, _BN, HALF), lambda i: (0, i, 0)),
                  pl.BlockSpec((NC, _BN, HALF), lambda i: (0, i, 0)),
                  pl.BlockSpec((3, C, C), lambda i: (0, 0, 0)),
                  pl.BlockSpec((C,), lambda i: (0,))],
        out_specs=[pl.BlockSpec((_BN, C), lambda i: (i, 0)),
                   pl.BlockSpec((2, C), lambda i: (0, 0))],
        out_shape=[jax.ShapeDtypeStruct((N, C), jnp.float32),
                   jax.ShapeDtypeStruct((2, C), jnp.float32)],
        scratch_shapes=[pltpu.VMEM((1, C), jnp.float32),
                        pltpu.VMEM((1, C), jnp.float32)],
    )(deg2, x, acc1, acc2, W, bias)


def _d2_body(st_ref, x_ref, y_ref, g_ref, bt_ref, o_ref):
    mean = st_ref[0] / N
    var = st_ref[1] / N - mean * mean
    scale = (lax.rsqrt(var + EPS) * g_ref[...])[None, :]
    o_ref[...] = jnp.maximum(
        (y_ref[...] - mean[None, :]) * scale + bt_ref[...][None, :] + x_ref[...],
        0.0)


def _d2_call(st, x, y, gamma, beta):
    return pl.pallas_call(
        _d2_body,
        grid=(_NB,),
        in_specs=[pl.BlockSpec((2, C), lambda i: (0, 0)),
                  pl.BlockSpec((_BN, C), lambda i: (i, 0)),
                  pl.BlockSpec((_BN, C), lambda i: (i, 0)),
                  pl.BlockSpec((C,), lambda i: (0,)),
                  pl.BlockSpec((C,), lambda i: (0,))],
        out_specs=pl.BlockSpec((_BN, C), lambda i: (i, 0)),
        out_shape=jax.ShapeDtypeStruct((N, C), jnp.float32),
    )(st, x, y, gamma, beta)


def kernel(x, edge_index, W, bias, gamma, beta):
    row = edge_index[0]
    col = edge_index[1]
    ept = E // NS
    ppt = _CPT * CH - ept
    rowp = jnp.pad(row.reshape(NS, ept), ((0, 0), (0, ppt))).reshape(-1)
    trash = jnp.broadcast_to(N + jnp.arange(ppt, dtype=jnp.int32), (NS, ppt))
    colp = jnp.concatenate([col.reshape(NS, ept), trash], axis=1).reshape(-1)
    degs = _deg_call(row)
    deg2 = degs[:, :N, 0].T
    hp0 = _e1_call(deg2, x).reshape(NC * NPAD, HALF)
    acc1 = _prop_call(hp0, rowp, colp)
    hp1 = _e2_call(deg2, acc1).reshape(NC * NPAD, HALF)
    acc2 = _prop_call(hp1, rowp, colp)
    y, st = _d1_call(deg2, x, acc1, acc2, W, bias)
    return _d2_call(st, x, y, gamma, beta)


# speedup vs baseline: 1.6735x; 1.5044x over previous
import functools

import jax
import jax.numpy as jnp
from jax import lax
from jax.experimental import pallas as pl
from jax.experimental.pallas import tpu as pltpu
from jax.experimental.pallas import tpu_sc as plsc

NC = 2
NS = 16
N = 10000
E = 160000
C = 256
HALF = C // 2
NPAD = 10240
ZPT = NPAD // NS
CH = 128
EPS = 1e-5

_MESH = plsc.VectorSubcoreMesh(core_axis_name="c", subcore_axis_name="s",
                               num_cores=NC, num_subcores=NS)


DW = 128


def _deg_body(rowi, ones_h, zrows, out, acc, ones_v, ones_t, ridx_v, ridx_t):
    c = lax.axis_index("c")
    s = lax.axis_index("s")
    pltpu.sync_copy(zrows, acc.at[pl.ds(s * ZPT, ZPT)])
    pltpu.sync_copy(ones_h, ones_v)
    pltpu.sync_copy(ones_h.at[pl.ds(0, 8)], ones_t)
    plsc.subcore_barrier()

    ept = E // (NC * NS)
    full = ept // CH
    tail = ept - full * CH
    ebase = (c * NS + s) * ept

    def do(off, ridx, ones, sz):
        pltpu.sync_copy(rowi.at[pl.ds(off, sz)], ridx)
        pltpu.sync_copy(ones, acc.at[ridx], add=True)

    def body(i, carry):
        do(ebase + i * CH, ridx_v, ones_v, CH)
        return carry

    lax.fori_loop(0, full, body, 0)
    do(ebase + full * CH, ridx_t, ones_t, tail)
    plsc.subcore_barrier()
    pltpu.sync_copy(acc.at[pl.ds(s * ZPT, ZPT)], out.at[c, pl.ds(s * ZPT, ZPT)])


def _deg_call(row):
    ones_h = jnp.ones((CH, DW), jnp.float32)
    zrows = jnp.zeros((ZPT, DW), jnp.float32)
    fn = pl.kernel(
        _deg_body,
        out_type=jax.ShapeDtypeStruct((NC, NPAD, DW), jnp.float32),
        mesh=_MESH,
        scratch_types=[
            pltpu.VMEM_SHARED((NPAD, DW), jnp.float32),
            pltpu.VMEM((CH, DW), jnp.float32),
            pltpu.VMEM((8, DW), jnp.float32),
            pltpu.VMEM((CH,), jnp.int32),
            pltpu.VMEM((8,), jnp.int32),
        ],
    )
    return fn(row, ones_h, zrows)


_GRP = 8
_NCHUNK = 1280
EPAD = _NCHUNK * CH
_CPT = _NCHUNK // NS
_NG = _CPT // _GRP


def _prop_body(rowi, coli, hp, zrows, out, acc,
               rows2, ridx2, cidx2, rows_t, ridx_t, cidx_t, gsem2, ssem2):
    c = lax.axis_index("c")
    s = lax.axis_index("s")
    coff = c * NPAD
    pltpu.sync_copy(zrows, acc.at[pl.ds(s * ZPT, ZPT)])
    plsc.subcore_barrier()

    ept = E // NS
    ebase = s * ept
    full = ept // CH
    tail = ept - full * CH
    npair = full // 2

    def loadidx(off, d):
        pltpu.sync_copy(rowi.at[pl.ds(off, CH)], ridx2[d])
        pltpu.sync_copy(coli.at[pl.ds(off, CH)], cidx2[d])
        for k in range(CH // 16):
            sl = pl.ds(k * 16, 16)
            ridx2[d][sl] = ridx2[d][sl] + coff

    for d in range(2):
        loadidx(ebase + d * CH, d)
        pltpu.async_copy(hp.at[ridx2[d]], rows2[d], gsem2[d])

    def body(j, carry):
        base_next = ebase + (j + 1) * 2 * CH
        for d in range(2):
            pltpu.make_async_copy(hp.at[ridx2[d]], rows2[d], gsem2[d]).wait()
            pltpu.async_copy(rows2[d], acc.at[cidx2[d]], ssem2[d], add=True)

        @pl.when(j < npair - 1)
        def _():
            for d in range(2):
                pltpu.make_async_copy(rows2[d], acc.at[cidx2[d]], ssem2[d]).wait()
                loadidx(base_next + d * CH, d)
                pltpu.async_copy(hp.at[ridx2[d]], rows2[d], gsem2[d])

        return carry

    lax.fori_loop(0, npair, body, 0)
    for d in range(2):
        pltpu.make_async_copy(rows2[d], acc.at[cidx2[d]], ssem2[d]).wait()

    toff = ebase + full * CH
    pltpu.sync_copy(rowi.at[pl.ds(toff, tail)], ridx_t)
    pltpu.sync_copy(coli.at[pl.ds(toff, tail)], cidx_t)
    ridx_t[pl.ds(0, 16)] = ridx_t[pl.ds(0, 16)] + coff
    pltpu.async_copy(hp.at[ridx_t], rows_t, gsem2[0]).wait()
    pltpu.sync_copy(rows_t, acc.at[cidx_t], add=True)

    plsc.subcore_barrier()
    pltpu.sync_copy(acc.at[pl.ds(s * ZPT, ZPT)], out.at[c, pl.ds(s * ZPT, ZPT)])


def _prop_call(hp_flat, rowp, colp):
    zrows = jnp.zeros((ZPT, HALF), jnp.float32)
    fn = pl.kernel(
        _prop_body,
        out_type=jax.ShapeDtypeStruct((NC, NPAD, HALF), jnp.float32),
        mesh=_MESH,
        scratch_types=[
            pltpu.VMEM_SHARED((NPAD, HALF), jnp.float32),
            [pltpu.VMEM((CH, HALF), jnp.float32)] * 2,
            [pltpu.VMEM((CH,), jnp.int32)] * 2,
            [pltpu.VMEM((CH,), jnp.int32)] * 2,
            pltpu.VMEM((16, HALF), jnp.float32),
            pltpu.VMEM((16,), jnp.int32),
            pltpu.VMEM((16,), jnp.int32),
            [pltpu.SemaphoreType.DMA] * 2,
            [pltpu.SemaphoreType.DMA] * 2,
        ],
    )
    return fn(rowp, colp, hp_flat, zrows)


_BN = 400
_NB = N // _BN


def _diw_block(deg_ref):
    deg = jnp.sum(deg_ref[...], axis=1, keepdims=True)
    return jnp.where(deg > 0, lax.rsqrt(jnp.maximum(deg, 1.0)), 0.0)


def _e1_body(deg_ref, x_ref, out_ref):
    d2 = _diw_block(deg_ref)
    out_ref[0] = d2 * x_ref[:, :HALF]
    out_ref[1] = d2 * x_ref[:, HALF:]


def _e1_call(deg2, x):
    return pl.pallas_call(
        _e1_body,
        grid=(_NB,),
        in_specs=[pl.BlockSpec((_BN, 2), lambda i: (i, 0)),
                  pl.BlockSpec((_BN, C), lambda i: (i, 0))],
        out_specs=pl.BlockSpec((NC, _BN, HALF), lambda i: (0, i, 0)),
        out_shape=jax.ShapeDtypeStruct((NC, NPAD, HALF), jnp.float32),
    )(deg2, x)


def _e2_body(deg_ref, a_ref, out_ref):
    deg = jnp.sum(deg_ref[...], axis=1, keepdims=True)
    w = jnp.where(deg > 0, -1.0 / jnp.maximum(deg, 1.0), 0.0)
    out_ref[0] = w * a_ref[0]
    out_ref[1] = w * a_ref[1]


def _e2_call(deg2, acc1):
    return pl.pallas_call(
        _e2_body,
        grid=(_NB,),
        in_specs=[pl.BlockSpec((_BN, 2), lambda i: (i, 0)),
                  pl.BlockSpec((NC, _BN, HALF), lambda i: (0, i, 0))],
        out_specs=pl.BlockSpec((NC, _BN, HALF), lambda i: (0, i, 0)),
        out_shape=jax.ShapeDtypeStruct((NC, NPAD, HALF), jnp.float32),
    )(deg2, acc1)


def _d1_body(deg_ref, x_ref, a1_ref, a2_ref, w_ref, b_ref,
             y_ref, st_ref, s_acc, q_acc):
    i = pl.program_id(0)
    d2 = _diw_block(deg_ref)
    x = x_ref[...]
    t1 = -d2 * jnp.concatenate([a1_ref[0], a1_ref[1]], axis=1)
    t2 = -2.0 * d2 * jnp.concatenate([a2_ref[0], a2_ref[1]], axis=1) - x
    y = jnp.dot(x, w_ref[0], preferred_element_type=jnp.float32)
    y = y + jnp.dot(t1, w_ref[1], preferred_element_type=jnp.float32)
    y = y + jnp.dot(t2, w_ref[2], preferred_element_type=jnp.float32)
    y = y + b_ref[...][None, :]
    y_ref[...] = y

    @pl.when(i == 0)
    def _():
        s_acc[...] = jnp.zeros_like(s_acc)
        q_acc[...] = jnp.zeros_like(q_acc)

    s_acc[...] += jnp.sum(y, axis=0, keepdims=True)
    q_acc[...] += jnp.sum(y * y, axis=0, keepdims=True)

    @pl.when(i == _NB - 1)
    def _():
        st_ref[0] = s_acc[0]
        st_ref[1] = q_acc[0]


def _d1_call(deg2, x, acc1, acc2, W, bias):
    return pl.pallas_call(
        _d1_body,
        grid=(_NB,),
        in_specs=[pl.BlockSpec((_BN, 2), lambda i: (i, 0)),
                  pl.BlockSpec((_BN, C), lambda i: (i, 0)),
                  pl.BlockSpec((NC, _BN, HALF), lambda i: (0, i, 0)),
                  pl.BlockSpec((NC, _BN, HALF), lambda i: (0, i, 0)),
                  pl.BlockSpec((3, C, C), lambda i: (0, 0, 0)),
                  pl.BlockSpec((C,), lambda i: (0,))],
        out_specs=[pl.BlockSpec((_BN, C), lambda i: (i, 0)),
                   pl.BlockSpec((2, C), lambda i: (0, 0))],
        out_shape=[jax.ShapeDtypeStruct((N, C), jnp.float32),
                   jax.ShapeDtypeStruct((2, C), jnp.float32)],
        scratch_shapes=[pltpu.VMEM((1, C), jnp.float32),
                        pltpu.VMEM((1, C), jnp.float32)],
    )(deg2, x, acc1, acc2, W, bias)


def _d2_body(st_ref, x_ref, y_ref, g_ref, bt_ref, o_ref):
    mean = st_ref[0] / N
    var = st_ref[1] / N - mean * mean
    scale = (lax.rsqrt(var + EPS) * g_ref[...])[None, :]
    o_ref[...] = jnp.maximum(
        (y_ref[...] - mean[None, :]) * scale + bt_ref[...][None, :] + x_ref[...],
        0.0)


def _d2_call(st, x, y, gamma, beta):
    return pl.pallas_call(
        _d2_body,
        grid=(_NB,),
        in_specs=[pl.BlockSpec((2, C), lambda i: (0, 0)),
                  pl.BlockSpec((_BN, C), lambda i: (i, 0)),
                  pl.BlockSpec((_BN, C), lambda i: (i, 0)),
                  pl.BlockSpec((C,), lambda i: (0,)),
                  pl.BlockSpec((C,), lambda i: (0,))],
        out_specs=pl.BlockSpec((_BN, C), lambda i: (i, 0)),
        out_shape=jax.ShapeDtypeStruct((N, C), jnp.float32),
    )(st, x, y, gamma, beta)


def kernel(x, edge_index, W, bias, gamma, beta):
    row = edge_index[0]
    col = edge_index[1]
    rowp = row
    colp = col
    degs = _deg_call(row)
    deg2 = degs[:, :N, 0].T
    hp0 = _e1_call(deg2, x).reshape(NC * NPAD, HALF)
    acc1 = _prop_call(hp0, rowp, colp)
    hp1 = _e2_call(deg2, acc1).reshape(NC * NPAD, HALF)
    acc2 = _prop_call(hp1, rowp, colp)
    y, st = _d1_call(deg2, x, acc1, acc2, W, bias)
    return _d2_call(st, x, y, gamma, beta)
